# baseline (speedup 1.0000x reference)
import functools

import jax
import jax.numpy as jnp
from jax import lax
from jax.experimental import pallas as pl
from jax.experimental.pallas import tpu as pltpu
from jax.experimental.pallas import tpu_sc as plsc

B = 16384
D = 32
NC = 2
NS = 16
NW = NC * NS
BPW = B // NW
NG = BPW // 16


def _body(u_hbm, i_hbm, f_hbm, ub_hbm, uvt_hbm, ib_hbm, ivt_hbm, par_hbm,
          out_hbm,
          u_v, i_v, f_v, bu_v, bi_v, vut_v, vit_v, par_v, out_v, sem):
    wid = lax.axis_index("c") * NS + lax.axis_index("s")
    base = wid * BPW

    pltpu.sync_copy(u_hbm.at[pl.ds(base, BPW)], u_v)
    pltpu.sync_copy(i_hbm.at[pl.ds(base, BPW)], i_v)
    pltpu.sync_copy(f_hbm.at[pl.ds(base, BPW)], f_v)
    pltpu.sync_copy(par_hbm, par_v)

    cps = []
    cps.append(pltpu.async_copy(ub_hbm.at[u_v], bu_v, sem))
    cps.append(pltpu.async_copy(ib_hbm.at[i_v], bi_v, sem))
    for d in range(D):
        cps.append(pltpu.async_copy(uvt_hbm.at[d].at[u_v], vut_v.at[d], sem))
        cps.append(pltpu.async_copy(ivt_hbm.at[d].at[i_v], vit_v.at[d], sem))
    for c in cps:
        c.wait()

    pv = par_v[pl.ds(0, 16)]
    c0 = pv[0]
    w0 = pv[1]
    w1 = pv[2]

    def body2(g, carry):
        rowbase = g * 16
        acc = vut_v[0, pl.ds(rowbase, 16)] * vit_v[0, pl.ds(rowbase, 16)]
        for d in range(1, D):
            acc = acc + vut_v[d, pl.ds(rowbase, 16)] * vit_v[d, pl.ds(rowbase, 16)]
        fv = f_v[pl.ds(rowbase, 16)]
        res = (acc + bu_v[pl.ds(rowbase, 16)] + bi_v[pl.ds(rowbase, 16)]
               + fv * w0 + fv * fv * w1 + c0)
        out_v[pl.ds(rowbase, 16)] = res
        return carry

    lax.fori_loop(0, NG, body2, 0)

    pltpu.sync_copy(out_v, out_hbm.at[pl.ds(base, BPW)])


_mf = functools.partial(
    pl.kernel,
    out_type=jax.ShapeDtypeStruct((B,), jnp.float32),
    mesh=plsc.VectorSubcoreMesh(core_axis_name="c", subcore_axis_name="s"),
    compiler_params=pltpu.CompilerParams(
        needs_layout_passes=False, use_tc_tiling_on_sc=False
    ),
    scratch_types=[
        pltpu.VMEM((BPW,), jnp.int32),
        pltpu.VMEM((BPW,), jnp.int32),
        pltpu.VMEM((BPW,), jnp.float32),
        pltpu.VMEM((BPW,), jnp.float32),
        pltpu.VMEM((BPW,), jnp.float32),
        pltpu.VMEM((D, BPW), jnp.float32),
        pltpu.VMEM((D, BPW), jnp.float32),
        pltpu.VMEM((16,), jnp.float32),
        pltpu.VMEM((BPW,), jnp.float32),
        pltpu.SemaphoreType.DMA,
    ],
)(_body)


def kernel(u, i, f, user_bias, user_vec, item_bias, item_vec, glob_bias,
           frame_w, frame_b):
    params = jnp.concatenate([
        glob_bias + frame_b,
        frame_w.reshape(2),
        jnp.zeros((13,), jnp.float32),
    ])
    return _mf(u.astype(jnp.int32), i.astype(jnp.int32), f,
               user_bias, user_vec.T, item_bias, item_vec.T, params)


# speedup vs baseline: 19.5883x; 19.5883x over previous
import functools

import jax
import jax.numpy as jnp
from jax import lax
from jax.experimental import pallas as pl
from jax.experimental.pallas import tpu as pltpu
from jax.experimental.pallas import tpu_sc as plsc

B = 16384
D = 32
V = 1000000
NC = 2
NS = 16
NW = NC * NS
BPW = B // NW
NG = BPW // 16

VT = 7813
FT = 4
ROWS = FT * VT * 8
CGRID = 13
CCH = 601 * 128
CROWS = 8 * 601


def _relin_body(src_ref, dst_ref):
    dst_ref[...] = src_ref[0].reshape(CROWS, 128)


def _relinearize(vt3):
    return pl.pallas_call(
        _relin_body,
        out_shape=jax.ShapeDtypeStruct((ROWS, 128), jnp.float32),
        grid=(FT, CGRID),
        in_specs=[pl.BlockSpec((1, 8, CCH), lambda ft, c: (ft, 0, c))],
        out_specs=pl.BlockSpec((CROWS, 128), lambda ft, c: (ft * CGRID + c, 0)),
    )(vt3)


def _mf_body(u_hbm, i_hbm, f_hbm, ub_hbm, uvl_hbm, ib_hbm, ivl_hbm, par_hbm,
             out_hbm,
             u_v, i_v, f_v, bu_v, bi_v, vut_v, vit_v, ou_v, oi_v, par_v,
             out_v, sem):
    wid = lax.axis_index("c") * NS + lax.axis_index("s")
    base = wid * BPW

    pltpu.sync_copy(u_hbm.at[pl.ds(base, BPW)], u_v)
    pltpu.sync_copy(i_hbm.at[pl.ds(base, BPW)], i_v)
    pltpu.sync_copy(f_hbm.at[pl.ds(base, BPW)], f_v)
    pltpu.sync_copy(par_hbm, par_v)

    def offsets(idx_v, off_ref):
        def obody(k, carry):
            v = idx_v[pl.ds(k * 16, 16)]
            c = (v.astype(jnp.float32) * (1.0 / 76928.0)
                 + 6e-6).astype(jnp.int32)
            rem = v - c * 76928
            bse = c * 615424 + ((rem >> 7) << 7) + (v & 127)
            for d in range(D):
                doff = (d >> 3) * 8000512 + (d & 7) * 76928
                off_ref[pl.ds(d * BPW + k * 16, 16)] = bse + doff
            return carry
        lax.fori_loop(0, NG, obody, 0)

    offsets(u_v, ou_v)
    offsets(i_v, oi_v)

    cps = [
        pltpu.async_copy(ub_hbm.at[u_v], bu_v, sem),
        pltpu.async_copy(ib_hbm.at[i_v], bi_v, sem),
    ]
    for d in range(D):
        cps.append(pltpu.async_copy(
            uvl_hbm.at[ou_v.at[pl.ds(d * BPW, BPW)]], vut_v.at[d], sem))
        cps.append(pltpu.async_copy(
            ivl_hbm.at[oi_v.at[pl.ds(d * BPW, BPW)]], vit_v.at[d], sem))
    for c in cps:
        c.wait()

    pv = par_v[pl.ds(0, 16)]
    c0 = pv[0]
    w0 = pv[1]
    w1 = pv[2]

    def body2(g, carry):
        rowbase = g * 16
        acc = vut_v[0, pl.ds(rowbase, 16)] * vit_v[0, pl.ds(rowbase, 16)]
        for d in range(1, D):
            acc = acc + vut_v[d, pl.ds(rowbase, 16)] * vit_v[d, pl.ds(rowbase, 16)]
        fv = f_v[pl.ds(rowbase, 16)]
        res = (acc + bu_v[pl.ds(rowbase, 16)] + bi_v[pl.ds(rowbase, 16)]
               + fv * w0 + fv * fv * w1 + c0)
        out_v[pl.ds(rowbase, 16)] = res
        return carry

    lax.fori_loop(0, NG, body2, 0)

    pltpu.sync_copy(out_v, out_hbm.at[pl.ds(base, BPW)])


_mf = functools.partial(
    pl.kernel,
    out_type=jax.ShapeDtypeStruct((B,), jnp.float32),
    mesh=plsc.VectorSubcoreMesh(core_axis_name="c", subcore_axis_name="s"),
    compiler_params=pltpu.CompilerParams(
        needs_layout_passes=False, use_tc_tiling_on_sc=False
    ),
    scratch_types=[
        pltpu.VMEM((BPW,), jnp.int32),
        pltpu.VMEM((BPW,), jnp.int32),
        pltpu.VMEM((BPW,), jnp.float32),
        pltpu.VMEM((BPW,), jnp.float32),
        pltpu.VMEM((BPW,), jnp.float32),
        pltpu.VMEM((D, BPW), jnp.float32),
        pltpu.VMEM((D, BPW), jnp.float32),
        pltpu.VMEM((D * BPW,), jnp.int32),
        pltpu.VMEM((D * BPW,), jnp.int32),
        pltpu.VMEM((16,), jnp.float32),
        pltpu.VMEM((BPW,), jnp.float32),
        pltpu.SemaphoreType.DMA,
    ],
)(_mf_body)


def kernel(u, i, f, user_bias, user_vec, item_bias, item_vec, glob_bias,
           frame_w, frame_b):
    params = jnp.concatenate([
        glob_bias + frame_b,
        frame_w.reshape(2),
        jnp.zeros((13,), jnp.float32),
    ])
    uvl = _relinearize(user_vec.T.reshape(FT, 8, V)).reshape(ROWS * 128)
    ivl = _relinearize(item_vec.T.reshape(FT, 8, V)).reshape(ROWS * 128)
    return _mf(u.astype(jnp.int32), i.astype(jnp.int32), f,
               user_bias, uvl, item_bias, ivl, params)


# speedup vs baseline: 22.0663x; 1.1265x over previous
import functools

import jax
import jax.numpy as jnp
from jax import lax
from jax.experimental import pallas as pl
from jax.experimental.pallas import tpu as pltpu
from jax.experimental.pallas import tpu_sc as plsc

B = 16384
D = 32
V = 1000000
NC = 2
NS = 16
NW = NC * NS
BPW = B // NW
NG = BPW // 16

FT = 4
CGRID = 8
CCH = 1024 * 128
CROWS = 8 * 1024
ROWS = FT * CGRID * CROWS


def _relin_body(src_ref, dst_ref):
    dst_ref[...] = src_ref[0].reshape(CROWS, 128)


def _relinearize(vt3):
    return pl.pallas_call(
        _relin_body,
        out_shape=jax.ShapeDtypeStruct((ROWS, 128), jnp.float32),
        grid=(FT, CGRID),
        in_specs=[pl.BlockSpec((1, 8, CCH), lambda ft, c: (ft, 0, c))],
        out_specs=pl.BlockSpec((CROWS, 128), lambda ft, c: (ft * CGRID + c, 0)),
    )(vt3)


def _mf_body(u_hbm, i_hbm, f_hbm, ub_hbm, uvl_hbm, ib_hbm, ivl_hbm, par_hbm,
             out_hbm,
             u_v, i_v, f_v, bu_v, bi_v, vut_v, vit_v, ou_v, oi_v, par_v,
             out_v, sem):
    wid = lax.axis_index("c") * NS + lax.axis_index("s")
    base = wid * BPW

    pltpu.sync_copy(u_hbm.at[pl.ds(base, BPW)], u_v)
    pltpu.sync_copy(i_hbm.at[pl.ds(base, BPW)], i_v)
    pltpu.sync_copy(f_hbm.at[pl.ds(base, BPW)], f_v)
    pltpu.sync_copy(par_hbm, par_v)

    def offsets(idx_v, off_ref):
        def obody(k, carry):
            v = idx_v[pl.ds(k * 16, 16)]
            bse = ((v >> 16) << 20) + (((v >> 7) & 511) << 7) + (v & 127)
            for d in range(D):
                doff = (d >> 3) * 8388608 + (d & 7) * 131072
                off_ref[pl.ds(d * BPW + k * 16, 16)] = bse + doff
            return carry
        lax.fori_loop(0, NG, obody, 0)

    offsets(u_v, ou_v)
    offsets(i_v, oi_v)

    cps = [
        pltpu.async_copy(ub_hbm.at[u_v], bu_v, sem),
        pltpu.async_copy(ib_hbm.at[i_v], bi_v, sem),
    ]
    for d in range(D):
        cps.append(pltpu.async_copy(
            uvl_hbm.at[ou_v.at[pl.ds(d * BPW, BPW)]], vut_v.at[d], sem))
        cps.append(pltpu.async_copy(
            ivl_hbm.at[oi_v.at[pl.ds(d * BPW, BPW)]], vit_v.at[d], sem))
    for c in cps:
        c.wait()

    pv = par_v[pl.ds(0, 16)]
    c0 = pv[0]
    w0 = pv[1]
    w1 = pv[2]

    def body2(g, carry):
        rowbase = g * 16
        acc = vut_v[0, pl.ds(rowbase, 16)] * vit_v[0, pl.ds(rowbase, 16)]
        for d in range(1, D):
            acc = acc + vut_v[d, pl.ds(rowbase, 16)] * vit_v[d, pl.ds(rowbase, 16)]
        fv = f_v[pl.ds(rowbase, 16)]
        res = (acc + bu_v[pl.ds(rowbase, 16)] + bi_v[pl.ds(rowbase, 16)]
               + fv * w0 + fv * fv * w1 + c0)
        out_v[pl.ds(rowbase, 16)] = res
        return carry

    lax.fori_loop(0, NG, body2, 0)

    pltpu.sync_copy(out_v, out_hbm.at[pl.ds(base, BPW)])


_mf = functools.partial(
    pl.kernel,
    out_type=jax.ShapeDtypeStruct((B,), jnp.float32),
    mesh=plsc.VectorSubcoreMesh(core_axis_name="c", subcore_axis_name="s"),
    compiler_params=pltpu.CompilerParams(
        needs_layout_passes=False, use_tc_tiling_on_sc=False
    ),
    scratch_types=[
        pltpu.VMEM((BPW,), jnp.int32),
        pltpu.VMEM((BPW,), jnp.int32),
        pltpu.VMEM((BPW,), jnp.float32),
        pltpu.VMEM((BPW,), jnp.float32),
        pltpu.VMEM((BPW,), jnp.float32),
        pltpu.VMEM((D, BPW), jnp.float32),
        pltpu.VMEM((D, BPW), jnp.float32),
        pltpu.VMEM((D * BPW,), jnp.int32),
        pltpu.VMEM((D * BPW,), jnp.int32),
        pltpu.VMEM((16,), jnp.float32),
        pltpu.VMEM((BPW,), jnp.float32),
        pltpu.SemaphoreType.DMA,
    ],
)(_mf_body)


def kernel(u, i, f, user_bias, user_vec, item_bias, item_vec, glob_bias,
           frame_w, frame_b):
    params = jnp.concatenate([
        glob_bias + frame_b,
        frame_w.reshape(2),
        jnp.zeros((13,), jnp.float32),
    ])
    uvl = _relinearize(user_vec.T.reshape(FT, 8, V)).reshape(ROWS * 128)
    ivl = _relinearize(item_vec.T.reshape(FT, 8, V)).reshape(ROWS * 128)
    return _mf(u.astype(jnp.int32), i.astype(jnp.int32), f,
               user_bias, uvl, item_bias, ivl, params)


# speedup vs baseline: 23.0117x; 1.0428x over previous
import functools

import jax
import jax.numpy as jnp
from jax import lax
from jax.experimental import pallas as pl
from jax.experimental.pallas import tpu as pltpu
from jax.experimental.pallas import tpu_sc as plsc

B = 16384
D = 32
V = 1000000
NC = 2
NS = 16
NW = NC * NS
BPW = B // NW
NG = BPW // 16

FT = 4
CGRID = 8
CCH = 1024 * 128
CROWS = 8 * 1024
ROWS = FT * CGRID * CROWS


def _relin_body(src_ref, dst_ref):
    dst_ref[...] = src_ref[0].reshape(CROWS, 128)


def _relinearize(vt3):
    return pl.pallas_call(
        _relin_body,
        out_shape=jax.ShapeDtypeStruct((ROWS, 128), jnp.float32),
        grid=(FT, CGRID),
        in_specs=[pl.BlockSpec((1, 8, CCH), lambda ft, c: (ft, 0, c))],
        out_specs=pl.BlockSpec((CROWS, 128), lambda ft, c: (ft * CGRID + c, 0)),
    )(vt3)


def _offsets(idx_v, off_ref):
    def obody(k, carry):
        v = idx_v[pl.ds(k * 16, 16)]
        bse = ((v >> 17) << 20) + (((v >> 7) & 1023) << 7) + (v & 127)
        for d in range(D):
            doff = (d >> 3) * 8388608 + (d & 7) * 131072
            off_ref[pl.ds(d * BPW + k * 16, 16)] = bse + doff
        return carry
    lax.fori_loop(0, NG, obody, 0)


def _ga_body(u_hbm, ub_hbm, uvl_hbm, vu_hbm, bu_hbm,
             u_v, bu_v, vut_v, ou_v, sem):
    wid = lax.axis_index("c") * NS + lax.axis_index("s")
    base = wid * BPW
    pltpu.sync_copy(u_hbm.at[pl.ds(base, BPW)], u_v)
    _offsets(u_v, ou_v)
    plsc.subcore_barrier()
    cps = [pltpu.async_copy(ub_hbm.at[u_v], bu_v, sem)]
    for d in range(D):
        cps.append(pltpu.async_copy(
            uvl_hbm.at[ou_v.at[pl.ds(d * BPW, BPW)]], vut_v.at[d], sem))
    for c in cps:
        c.wait()
    for d in range(D):
        pltpu.sync_copy(vut_v.at[d],
                        vu_hbm.at[pl.ds((wid * D + d) * BPW, BPW)])
    pltpu.sync_copy(bu_v, bu_hbm.at[pl.ds(base, BPW)])


_ga = functools.partial(
    pl.kernel,
    out_type=(
        jax.ShapeDtypeStruct((NW * D * BPW,), jnp.float32),
        jax.ShapeDtypeStruct((B,), jnp.float32),
    ),
    mesh=plsc.VectorSubcoreMesh(core_axis_name="c", subcore_axis_name="s"),
    compiler_params=pltpu.CompilerParams(
        needs_layout_passes=False, use_tc_tiling_on_sc=False
    ),
    scratch_types=[
        pltpu.VMEM((BPW,), jnp.int32),
        pltpu.VMEM((BPW,), jnp.float32),
        pltpu.VMEM((D, BPW), jnp.float32),
        pltpu.VMEM((D * BPW,), jnp.int32),
        pltpu.SemaphoreType.DMA,
    ],
)(_ga_body)


def _mf_body(i_hbm, f_hbm, ib_hbm, ivl_hbm, vu_hbm, bu_hbm, par_hbm,
             out_hbm,
             i_v, f_v, bu_v, bi_v, vut_v, vit_v, oi_v, par_v, out_v, sem):
    wid = lax.axis_index("c") * NS + lax.axis_index("s")
    base = wid * BPW

    pltpu.sync_copy(i_hbm.at[pl.ds(base, BPW)], i_v)
    pltpu.sync_copy(f_hbm.at[pl.ds(base, BPW)], f_v)
    pltpu.sync_copy(par_hbm, par_v)
    _offsets(i_v, oi_v)

    cps = [
        pltpu.async_copy(ib_hbm.at[i_v], bi_v, sem),
    ]
    for d in range(D):
        cps.append(pltpu.async_copy(
            vu_hbm.at[pl.ds((wid * D + d) * BPW, BPW)], vut_v.at[d], sem))
    cps += [
        pltpu.async_copy(bu_hbm.at[pl.ds(base, BPW)], bu_v, sem),
    ]
    for d in range(D):
        cps.append(pltpu.async_copy(
            ivl_hbm.at[oi_v.at[pl.ds(d * BPW, BPW)]], vit_v.at[d], sem))
    for c in cps:
        c.wait()

    pv = par_v[pl.ds(0, 16)]
    c0 = pv[0]
    w0 = pv[1]
    w1 = pv[2]

    def body2(g, carry):
        rowbase = g * 16
        acc = vut_v[0, pl.ds(rowbase, 16)] * vit_v[0, pl.ds(rowbase, 16)]
        for d in range(1, D):
            acc = (acc + vut_v[d, pl.ds(rowbase, 16)]
                   * vit_v[d, pl.ds(rowbase, 16)])
        fv = f_v[pl.ds(rowbase, 16)]
        res = (acc + bu_v[pl.ds(rowbase, 16)] + bi_v[pl.ds(rowbase, 16)]
               + fv * w0 + fv * fv * w1 + c0)
        out_v[pl.ds(rowbase, 16)] = res
        return carry

    lax.fori_loop(0, NG, body2, 0)

    pltpu.sync_copy(out_v, out_hbm.at[pl.ds(base, BPW)])


_mf = functools.partial(
    pl.kernel,
    out_type=jax.ShapeDtypeStruct((B,), jnp.float32),
    mesh=plsc.VectorSubcoreMesh(core_axis_name="c", subcore_axis_name="s"),
    compiler_params=pltpu.CompilerParams(
        needs_layout_passes=False, use_tc_tiling_on_sc=False
    ),
    scratch_types=[
        pltpu.VMEM((BPW,), jnp.int32),
        pltpu.VMEM((BPW,), jnp.float32),
        pltpu.VMEM((BPW,), jnp.float32),
        pltpu.VMEM((BPW,), jnp.float32),
        pltpu.VMEM((D, BPW), jnp.float32),
        pltpu.VMEM((D, BPW), jnp.float32),
        pltpu.VMEM((D * BPW,), jnp.int32),
        pltpu.VMEM((16,), jnp.float32),
        pltpu.VMEM((BPW,), jnp.float32),
        pltpu.SemaphoreType.DMA,
    ],
)(_mf_body)


def kernel(u, i, f, user_bias, user_vec, item_bias, item_vec, glob_bias,
           frame_w, frame_b):
    params = jnp.concatenate([
        glob_bias + frame_b,
        frame_w.reshape(2),
        jnp.zeros((13,), jnp.float32),
    ])
    uvl = _relinearize(user_vec.T.reshape(FT, 8, V)).reshape(ROWS * 128)
    vu, bu = _ga(u.astype(jnp.int32), user_bias, uvl)
    ivl = _relinearize(item_vec.T.reshape(FT, 8, V)).reshape(ROWS * 128)
    return _mf(i.astype(jnp.int32), f, item_bias, ivl, vu, bu, params)
